# baseline (speedup 1.0000x reference)
import functools

import jax
import jax.numpy as jnp
from jax import lax
from jax.experimental import pallas as pl
from jax.experimental.pallas import tpu as pltpu
from jax.experimental.pallas import tpu_sc as plsc

NC = 2
NS = 16
NW = NC * NS


def _pick_cw(n_main):
    best = 128
    for t in range(1, n_main // 128 + 1):
        cw = 128 * t
        if cw > 5000:
            break
        if n_main % cw == 0:
            best = cw
    return best


def _row_minmax_sc(adjacency, n_main, row_start):
    n_rows = adjacency.shape[0]
    n_sc = n_rows - row_start
    RW = ((n_sc + NW * 16 - 1) // (NW * 16)) * 16
    NPAD = NW * RW
    GROUPS = RW // 16
    CW = _pick_cw(n_main)
    NCC = n_main // CW
    JMAX = CW // 16

    mesh = plsc.VectorSubcoreMesh(
        core_axis_name="c", subcore_axis_name="s",
        num_cores=NC, num_subcores=NS,
    )

    @functools.partial(
        pl.kernel,
        out_type=[
            jax.ShapeDtypeStruct((NPAD,), jnp.float32),
            jax.ShapeDtypeStruct((NPAD,), jnp.float32),
        ],
        mesh=mesh,
        compiler_params=pltpu.CompilerParams(needs_layout_passes=False),
        scratch_types=[
            pltpu.VMEM((8, CW), jnp.float32),
            pltpu.VMEM((8, CW), jnp.float32),
            pltpu.VMEM((16, 17), jnp.float32),
            pltpu.VMEM((16, 17), jnp.float32),
            pltpu.VMEM((16,), jnp.float32),
            pltpu.VMEM((16,), jnp.float32),
            pltpu.SemaphoreType.DMA,
            pltpu.SemaphoreType.DMA,
        ],
    )
    def rowminmax(adj_hbm, rmax_hbm, rmin_hbm, buf0, buf1, trmax, trmin,
                  stg_max, stg_min, sem0, sem1):
        wid = lax.axis_index("s") * NC + lax.axis_index("c")
        base = row_start + wid * RW
        lane = lax.iota(jnp.int32, 16)

        def do_group(g, _):
            rb = base + 16 * g

            @pl.when(rb < n_rows)
            def _():
                bufs = (buf0, buf1)
                sems = (sem0, sem1)
                Q = 2 * NCC

                def src(q):
                    return adj_hbm.at[pl.ds(rb + 8 * (q // NCC), 8),
                                      pl.ds((q % NCC) * CW, CW)]

                copies = {}
                for q in range(min(2, Q)):
                    copies[q] = pltpu.async_copy(src(q), bufs[q % 2],
                                                 sems[q % 2])

                for q in range(Q):
                    copies[q].wait()
                    nxt = q + 2
                    if nxt < Q:
                        copies[nxt] = pltpu.async_copy(
                            src(nxt), bufs[nxt % 2], sems[nxt % 2])
                    buf = bufs[q % 2]
                    rbase = 8 * (q // NCC)

                    def rstep(r, _, first=(q % NCC == 0), rbase=rbase):
                        def jstep(j, acc):
                            am, an = acc
                            v = buf[r, pl.ds(j * 16, 16)]
                            return jnp.maximum(am, v), jnp.minimum(an, v)

                        am0 = jnp.full((16,), -jnp.inf, jnp.float32)
                        an0 = jnp.full((16,), jnp.inf, jnp.float32)
                        am, an = plsc.parallel_loop(
                            0, JMAX, carry=(am0, an0), unroll=8)(jstep)
                        tr = rbase + r
                        if not first:
                            am = jnp.maximum(am, trmax[tr, pl.ds(0, 16)])
                            an = jnp.minimum(an, trmin[tr, pl.ds(0, 16)])
                        trmax[tr, pl.ds(0, 16)] = am
                        trmin[tr, pl.ds(0, 16)] = an
                        return 0

                    lax.fori_loop(0, 8, rstep, 0)

                gmax = jnp.full((16,), -jnp.inf, jnp.float32)
                gmin = jnp.full((16,), jnp.inf, jnp.float32)
                for j in range(16):
                    col = jnp.full((16,), j, jnp.int32)
                    gmax = jnp.maximum(gmax, plsc.load_gather(trmax, [lane, col]))
                    gmin = jnp.minimum(gmin, plsc.load_gather(trmin, [lane, col]))

                stg_max[...] = gmax
                stg_min[...] = gmin
                pltpu.sync_copy(stg_max, rmax_hbm.at[pl.ds(rb - row_start, 16)])
                pltpu.sync_copy(stg_min, rmin_hbm.at[pl.ds(rb - row_start, 16)])

            return 0

        lax.fori_loop(0, GROUPS, do_group, 0)

    rmax_pad, rmin_pad = rowminmax(adjacency)
    return rmax_pad[:n_sc], rmin_pad[:n_sc]


def _scan_combine_tc(adjacency, features, W, tail, n_tc, n_main):
    d = features.shape[1]
    tw = tail.shape[1]
    bm = 8
    for t in range(1, n_tc // 8 + 1):
        if t * 8 > 256:
            break
        if n_tc % (t * 8) == 0:
            bm = t * 8

    def body(a_ref, f_ref, w_ref, tail_ref, o_ref):
        blk = a_ref[...]
        t = tail_ref[...]
        rmx = jnp.maximum(jnp.max(blk, axis=1, keepdims=True),
                          jnp.max(t, axis=1, keepdims=True))
        rmn = jnp.minimum(jnp.min(blk, axis=1, keepdims=True),
                          jnp.min(t, axis=1, keepdims=True))
        out = jnp.dot(f_ref[...], w_ref[...],
                      preferred_element_type=jnp.float32)
        o_ref[...] = jnp.maximum(jnp.maximum(out * rmx, out * rmn), 0.0)

    n_rows = adjacency.shape[0]
    return pl.pallas_call(
        body,
        grid=(n_tc // bm,),
        compiler_params=pltpu.CompilerParams(
            dimension_semantics=("arbitrary",)),
        in_specs=[
            pl.BlockSpec((bm, n_main), lambda i: (i, 0)),
            pl.BlockSpec((bm, d), lambda i: (i, 0)),
            pl.BlockSpec((d, d), lambda i: (0, 0)),
            pl.BlockSpec((bm, tw), lambda i: (i, 0)),
        ],
        out_specs=pl.BlockSpec((bm, d), lambda i: (i, 0)),
        out_shape=jax.ShapeDtypeStruct((n_rows, d), jnp.float32),
    )(adjacency, features, W, tail)


def _combine_sc_rows(features, W, rmax, rmin, tail, agg, r_tc, bm):
    m, d = features.shape
    n_sc = m - r_tc
    tw = tail.shape[1]
    blk_off = r_tc // bm

    def body(f_ref, w_ref, rmm_ref, tail_ref, agg_ref, o_ref):
        del agg_ref
        out = jnp.dot(f_ref[...], w_ref[...],
                      preferred_element_type=jnp.float32)
        t = tail_ref[...]
        rmm = rmm_ref[...]
        rmx = jnp.maximum(rmm[0], jnp.max(t, axis=1, keepdims=True))
        rmn = jnp.minimum(rmm[1], jnp.min(t, axis=1, keepdims=True))
        o_ref[...] = jnp.maximum(jnp.maximum(out * rmx, out * rmn), 0.0)

    rmm = jnp.stack([rmax, rmin]).reshape(2, n_sc, 1)
    return pl.pallas_call(
        body,
        grid=(n_sc // bm,),
        compiler_params=pltpu.CompilerParams(
            dimension_semantics=("arbitrary",)),
        in_specs=[
            pl.BlockSpec((bm, d), lambda i: (i + blk_off, 0)),
            pl.BlockSpec((d, d), lambda i: (0, 0)),
            pl.BlockSpec((2, bm, 1), lambda i: (0, i, 0)),
            pl.BlockSpec((bm, tw), lambda i: (i + blk_off, 0)),
            pl.BlockSpec(memory_space=pl.ANY),
        ],
        out_specs=pl.BlockSpec((bm, d), lambda i: (i + blk_off, 0)),
        out_shape=jax.ShapeDtypeStruct((m, d), jnp.float32),
        input_output_aliases={4: 0},
    )(features, W, rmm, tail, agg)


@jax.jit
def kernel(features, adjacency, W):
    n_rows, n_cols = adjacency.shape
    n_main = (n_cols // 128) * 128
    if n_main == n_cols:
        n_main -= 128
    bm2 = 400
    n_sc = (n_rows * 44 // 100) // bm2 * bm2
    r_tc = n_rows - n_sc
    if n_sc == 0 or r_tc % bm2 or n_sc % 16:
        bm2 = 16
        n_sc = (n_rows * 40 // 100) // 16 * 16
        r_tc = n_rows - n_sc
    rmax_sc, rmin_sc = _row_minmax_sc(adjacency, n_main, r_tc)
    tail = adjacency[:, n_main:]
    agg = _scan_combine_tc(adjacency, features, W, tail, r_tc, n_main)
    return _combine_sc_rows(features, W, rmax_sc, rmin_sc, tail, agg,
                            r_tc, bm2)


# speedup vs baseline: 1.0476x; 1.0476x over previous
import functools

import jax
import jax.numpy as jnp
from jax import lax
from jax.experimental import pallas as pl
from jax.experimental.pallas import tpu as pltpu
from jax.experimental.pallas import tpu_sc as plsc

NC = 2
NS = 16
NW = NC * NS


def _pick_cw(n_main):
    best = 128
    for t in range(1, n_main // 128 + 1):
        cw = 128 * t
        if cw > 5000:
            break
        if n_main % cw == 0:
            best = cw
    return best


def _row_minmax_sc(adjacency, n_main, row_start):
    n_rows = adjacency.shape[0]
    n_sc = n_rows - row_start
    RW = ((n_sc + NW * 16 - 1) // (NW * 16)) * 16
    NPAD = NW * RW
    GROUPS = RW // 16
    CW = _pick_cw(n_main)
    NCC = n_main // CW
    JMAX = CW // 16

    mesh = plsc.VectorSubcoreMesh(
        core_axis_name="c", subcore_axis_name="s",
        num_cores=NC, num_subcores=NS,
    )

    @functools.partial(
        pl.kernel,
        out_type=[
            jax.ShapeDtypeStruct((NPAD,), jnp.float32),
            jax.ShapeDtypeStruct((NPAD,), jnp.float32),
        ],
        mesh=mesh,
        compiler_params=pltpu.CompilerParams(needs_layout_passes=False),
        scratch_types=[
            pltpu.VMEM((8, CW), jnp.float32),
            pltpu.VMEM((8, CW), jnp.float32),
            pltpu.VMEM((16, 17), jnp.float32),
            pltpu.VMEM((16, 17), jnp.float32),
            pltpu.VMEM((16,), jnp.float32),
            pltpu.VMEM((16,), jnp.float32),
            pltpu.SemaphoreType.DMA,
            pltpu.SemaphoreType.DMA,
        ],
    )
    def rowminmax(adj_hbm, rmax_hbm, rmin_hbm, buf0, buf1, trmax, trmin,
                  stg_max, stg_min, sem0, sem1):
        wid = lax.axis_index("s") * NC + lax.axis_index("c")
        base = row_start + wid * RW
        lane = lax.iota(jnp.int32, 16)

        def do_group(g, _):
            rb = base + 16 * g

            @pl.when(rb < n_rows)
            def _():
                bufs = (buf0, buf1)
                sems = (sem0, sem1)
                Q = 2 * NCC

                def src(q):
                    return adj_hbm.at[pl.ds(rb + 8 * (q // NCC), 8),
                                      pl.ds((q % NCC) * CW, CW)]

                copies = {}
                for q in range(min(2, Q)):
                    copies[q] = pltpu.async_copy(src(q), bufs[q % 2],
                                                 sems[q % 2])

                for q in range(Q):
                    copies[q].wait()
                    nxt = q + 2
                    if nxt < Q:
                        copies[nxt] = pltpu.async_copy(
                            src(nxt), bufs[nxt % 2], sems[nxt % 2])
                    buf = bufs[q % 2]
                    rbase = 8 * (q // NCC)

                    def rstep(r, _, first=(q % NCC == 0), rbase=rbase):
                        def jstep(j, acc):
                            am, an = acc
                            v = buf[r, pl.ds(j * 16, 16)]
                            return jnp.maximum(am, v), jnp.minimum(an, v)

                        am0 = jnp.full((16,), -jnp.inf, jnp.float32)
                        an0 = jnp.full((16,), jnp.inf, jnp.float32)
                        am, an = plsc.parallel_loop(
                            0, JMAX, carry=(am0, an0), unroll=8)(jstep)
                        tr = rbase + r
                        if not first:
                            am = jnp.maximum(am, trmax[tr, pl.ds(0, 16)])
                            an = jnp.minimum(an, trmin[tr, pl.ds(0, 16)])
                        trmax[tr, pl.ds(0, 16)] = am
                        trmin[tr, pl.ds(0, 16)] = an
                        return 0

                    lax.fori_loop(0, 8, rstep, 0)

                gmax = jnp.full((16,), -jnp.inf, jnp.float32)
                gmin = jnp.full((16,), jnp.inf, jnp.float32)
                for j in range(16):
                    col = jnp.full((16,), j, jnp.int32)
                    gmax = jnp.maximum(gmax, plsc.load_gather(trmax, [lane, col]))
                    gmin = jnp.minimum(gmin, plsc.load_gather(trmin, [lane, col]))

                stg_max[...] = gmax
                stg_min[...] = gmin
                pltpu.sync_copy(stg_max, rmax_hbm.at[pl.ds(rb - row_start, 16)])
                pltpu.sync_copy(stg_min, rmin_hbm.at[pl.ds(rb - row_start, 16)])

            return 0

        lax.fori_loop(0, GROUPS, do_group, 0)

    rmax_pad, rmin_pad = rowminmax(adjacency)
    return rmax_pad[:n_sc], rmin_pad[:n_sc]


def _scan_combine_tc(adjacency, features, W, tail, n_tc, n_main):
    d = features.shape[1]
    tw = tail.shape[1]
    bm = 8
    for t in range(1, n_tc // 8 + 1):
        if t * 8 > 256:
            break
        if n_tc % (t * 8) == 0:
            bm = t * 8

    def body(a_ref, f_ref, w_ref, tail_ref, o_ref):
        blk = a_ref[...]
        t = tail_ref[...]
        rmx = jnp.maximum(jnp.max(blk, axis=1, keepdims=True),
                          jnp.max(t, axis=1, keepdims=True))
        rmn = jnp.minimum(jnp.min(blk, axis=1, keepdims=True),
                          jnp.min(t, axis=1, keepdims=True))
        out = jnp.dot(f_ref[...], w_ref[...],
                      preferred_element_type=jnp.float32)
        o_ref[...] = jnp.maximum(jnp.maximum(out * rmx, out * rmn), 0.0)

    n_rows = adjacency.shape[0]
    return pl.pallas_call(
        body,
        grid=(n_tc // bm,),
        compiler_params=pltpu.CompilerParams(
            dimension_semantics=("arbitrary",)),
        in_specs=[
            pl.BlockSpec((bm, n_main), lambda i: (i, 0)),
            pl.BlockSpec((bm, d), lambda i: (i, 0)),
            pl.BlockSpec((d, d), lambda i: (0, 0)),
            pl.BlockSpec((bm, tw), lambda i: (i, 0)),
        ],
        out_specs=pl.BlockSpec((bm, d), lambda i: (i, 0)),
        out_shape=jax.ShapeDtypeStruct((n_rows, d), jnp.float32),
    )(adjacency, features, W, tail)


def _combine_sc_rows(features, W, rmax, rmin, tail, agg, r_tc, bm):
    m, d = features.shape
    n_sc = m - r_tc
    tw = tail.shape[1]
    blk_off = r_tc // bm

    def body(f_ref, w_ref, rmm_ref, tail_ref, agg_ref, o_ref):
        del agg_ref
        out = jnp.dot(f_ref[...], w_ref[...],
                      preferred_element_type=jnp.float32)
        t = tail_ref[...]
        rmm = rmm_ref[...]
        rmx = jnp.maximum(rmm[0], jnp.max(t, axis=1, keepdims=True))
        rmn = jnp.minimum(rmm[1], jnp.min(t, axis=1, keepdims=True))
        o_ref[...] = jnp.maximum(jnp.maximum(out * rmx, out * rmn), 0.0)

    rmm = jnp.stack([rmax, rmin]).reshape(2, n_sc, 1)
    return pl.pallas_call(
        body,
        grid=(n_sc // bm,),
        compiler_params=pltpu.CompilerParams(
            dimension_semantics=("arbitrary",)),
        in_specs=[
            pl.BlockSpec((bm, d), lambda i: (i + blk_off, 0)),
            pl.BlockSpec((d, d), lambda i: (0, 0)),
            pl.BlockSpec((2, bm, 1), lambda i: (0, i, 0)),
            pl.BlockSpec((bm, tw), lambda i: (i + blk_off, 0)),
            pl.BlockSpec(memory_space=pl.ANY),
        ],
        out_specs=pl.BlockSpec((bm, d), lambda i: (i + blk_off, 0)),
        out_shape=jax.ShapeDtypeStruct((m, d), jnp.float32),
        input_output_aliases={4: 0},
    )(features, W, rmm, tail, agg)


@jax.jit
def kernel(features, adjacency, W):
    n_rows, n_cols = adjacency.shape
    n_main = (n_cols // 128) * 128
    if n_main == n_cols:
        n_main -= 128
    bm2 = 1000
    n_sc = (n_rows * 40 // 100) // bm2 * bm2
    r_tc = n_rows - n_sc
    if n_sc == 0 or r_tc % bm2 or n_sc % 16:
        bm2 = 16
        n_sc = (n_rows * 40 // 100) // 16 * 16
        r_tc = n_rows - n_sc
    rmax_sc, rmin_sc = _row_minmax_sc(adjacency, n_main, r_tc)
    tail = adjacency[:, n_main:]
    agg = _scan_combine_tc(adjacency, features, W, tail, r_tc, n_main)
    return _combine_sc_rows(features, W, rmax_sc, rmin_sc, tail, agg,
                            r_tc, bm2)
